# Initial kernel scaffold; baseline (speedup 1.0000x reference)
import jax
import jax.numpy as jnp
from jax.experimental import pallas as pl


def kernel(x, gamma):
    raise NotImplementedError("write your pallas kernel here")


# speedup vs baseline: 1.4178x; 1.4178x over previous
import numpy as np

import jax
import jax.numpy as jnp
from jax.experimental import pallas as pl
from jax.experimental.pallas import tpu as pltpu

_B, _C, _H, _W = 8, 192, 224, 224
_BS = 5
_HS, _WS = _H - (_BS - 1), _W - (_BS - 1)
_D = _B * _C
_COUNT_M = float(_D * _H * _W)
_SEEDS_PER_SAMPLE = _HS * _WS

_ROTS = ((13, 15, 26, 6), (17, 29, 16, 24))


def _threefry_key():
    def tf2x32(k0, k1, x0, x1):
        M = 0xFFFFFFFF
        ks = (k0, k1, 0x1BD11BDA ^ k0 ^ k1)
        x0 = (x0 + ks[0]) & M
        x1 = (x1 + ks[1]) & M
        for g in range(5):
            for r in _ROTS[g % 2]:
                x0 = (x0 + x1) & M
                x1 = ((x1 << r) | (x1 >> (32 - r))) & M
                x1 ^= x0
            x0 = (x0 + ks[(g + 1) % 3]) & M
            x1 = (x1 + ks[(g + 2) % 3] + g + 1) & M
        return x0, x1
    return tf2x32(0, 0, 0, 123)


_K0, _K1 = _threefry_key()
_K2 = 0x1BD11BDA ^ _K0 ^ _K1


def _random_bits(ctr):
    ks = (np.uint32(_K0), np.uint32(_K1), np.uint32(_K2))
    x0 = jnp.full(ctr.shape, ks[0], jnp.uint32)
    x1 = ctr + ks[1]
    for g in range(5):
        for r in _ROTS[g % 2]:
            x0 = x0 + x1
            x1 = (x1 << np.uint32(r)) | (x1 >> np.uint32(32 - r))
            x1 = x1 ^ x0
        x0 = x0 + ks[(g + 1) % 3]
        x1 = x1 + np.uint32((int(ks[(g + 2) % 3]) + g + 1) & 0xFFFFFFFF)
    return x0 ^ x1


def _shift2d(a, s, axis):
    if axis == 0:
        pad = jnp.zeros((s, a.shape[1]), a.dtype)
        return jnp.concatenate([pad, a[: a.shape[0] - s, :]], axis=0)
    pad = jnp.zeros((a.shape[0], s), a.dtype)
    return jnp.concatenate([pad, a[:, : a.shape[1] - s]], axis=1)


def _mask_kernel(gamma_ref, mask_ref, cnt_ref):
    i = pl.program_id(0)
    y = jax.lax.broadcasted_iota(jnp.uint32, (_H, _W), 0)
    xc = jax.lax.broadcasted_iota(jnp.uint32, (_H, _W), 1)
    base = i.astype(jnp.uint32) * np.uint32(_SEEDS_PER_SAMPLE)
    ctr = base + y * np.uint32(_WS) + xc
    bits = _random_bits(ctr)
    mb = (bits >> np.uint32(9)) | np.uint32(0x3F800000)
    f = jax.lax.bitcast_convert_type(mb, jnp.float32) - jnp.float32(1.0)
    gamma = gamma_ref[0, 0]
    valid = (y < np.uint32(_HS)) & (xc < np.uint32(_WS))
    seed = jnp.where((f < gamma) & valid, jnp.float32(1.0), jnp.float32(0.0))
    c = seed
    for s in (1, 2, 3, 4):
        c = jnp.maximum(c, _shift2d(seed, s, axis=1))
    d = c
    for s in (1, 2, 3, 4):
        d = jnp.maximum(d, _shift2d(c, s, axis=0))
    keep = jnp.float32(1.0) - d
    mask_ref[0] = keep.astype(jnp.int8)
    tile_ones = jnp.sum(keep).astype(jnp.int32)

    @pl.when(i == 0)
    def _init():
        cnt_ref[0, 0] = tile_ones

    @pl.when(i > 0)
    def _acc():
        cnt_ref[0, 0] = cnt_ref[0, 0] + tile_ones


_APPLY_BLK = 8


def _apply_kernel(cnt_ref, x_ref, mask_ref, o_ref):
    scale = jnp.float32(_COUNT_M) / cnt_ref[0, 0].astype(jnp.float32)
    o_ref[...] = x_ref[...] * (mask_ref[...].astype(jnp.float32) * scale)


def kernel(x, gamma):
    xr = x.reshape(_D, _H, _W)
    g2 = jnp.asarray(gamma, jnp.float32).reshape(1, 1)

    mask, cnt = pl.pallas_call(
        _mask_kernel,
        grid=(_D,),
        in_specs=[pl.BlockSpec(memory_space=pltpu.SMEM)],
        out_specs=[
            pl.BlockSpec((1, _H, _W), lambda i: (i, 0, 0)),
            pl.BlockSpec(memory_space=pltpu.SMEM),
        ],
        out_shape=[
            jax.ShapeDtypeStruct((_D, _H, _W), jnp.int8),
            jax.ShapeDtypeStruct((1, 1), jnp.int32),
        ],
    )(g2)

    out = pl.pallas_call(
        _apply_kernel,
        grid=(_D // _APPLY_BLK,),
        in_specs=[
            pl.BlockSpec(memory_space=pltpu.SMEM),
            pl.BlockSpec((_APPLY_BLK, _H, _W), lambda i: (i, 0, 0)),
            pl.BlockSpec((_APPLY_BLK, _H, _W), lambda i: (i, 0, 0)),
        ],
        out_specs=pl.BlockSpec((_APPLY_BLK, _H, _W), lambda i: (i, 0, 0)),
        out_shape=jax.ShapeDtypeStruct((_D, _H, _W), jnp.float32),
    )(cnt, xr, mask)

    return out.reshape(x.shape)
